# baseline (speedup 1.0000x reference)
import jax
import jax.numpy as jnp
from jax import lax
from jax.experimental import pallas as pl
from jax.experimental.pallas import tpu as pltpu
from jax.experimental.pallas import tpu_sc as plsc

_N, _C, _T, _HW = 8, 256, 32, 196
_ROW = _T * _HW
_NROWS = _N * _C
_NW = 32
_KR = 8
_CHUNKS_PER_W = _NROWS // (_NW * _KR)
_G = _ROW // 16
_GSEL = (_T - 1) * _HW // 16
_OUT1 = _NROWS * _ROW


def _sc_body(x_hbm, sel_hbm, out_hbm, buf, buf2, selv, sems):
    wid = lax.axis_index("s") * 2 + lax.axis_index("c")
    pltpu.sync_copy(sel_hbm, selv)
    wbase = wid * (_NROWS // _NW) * _ROW

    def chunk_step(k, carry):
        r0 = wbase + k * _KR * _ROW
        cin = pltpu.make_async_copy(
            x_hbm.at[pl.ds(r0, _KR * _ROW)],
            buf.at[pl.ds(0, _KR * _ROW)],
            sems.at[0],
        )
        cin.start()
        cin.wait()
        c0 = pltpu.make_async_copy(
            buf.at[pl.ds(0, _KR * _ROW)],
            out_hbm.at[pl.ds(r0, _KR * _ROW)],
            sems.at[1],
        )
        c0.start()

        def row_step(r, c):
            rb = r * _ROW
            for g in range(_G):
                off = rb + 16 * g
                a = buf[pl.ds(off, 16)]
                if g >= 380:
                    buf2[pl.ds(off, 16)] = a
                else:
                    b = buf[pl.ds(off + _HW, 16)]
                    s = selv[pl.ds(16 * g, 16)]
                    buf2[pl.ds(off, 16)] = jnp.where(s != 0, a, b)
            return c

        lax.fori_loop(0, _KR, row_step, 0)
        c1 = pltpu.make_async_copy(
            buf2,
            out_hbm.at[pl.ds(_OUT1 + r0, _KR * _ROW)],
            sems.at[2],
        )
        c1.start()
        c0.wait()
        c1.wait()
        return carry

    lax.fori_loop(0, _CHUNKS_PER_W, chunk_step, 0)


def kernel(x, t_length):
    N, C, T, H, W = x.shape
    idx = jnp.mod(t_length.astype(jnp.int32) - 1, T)
    keep = jnp.zeros((T,), jnp.int32).at[idx].set(1).at[T - 1].set(1)
    sel = jnp.repeat(keep, H * W)
    x_flat = x.reshape(-1)

    mesh = plsc.VectorSubcoreMesh(core_axis_name="c", subcore_axis_name="s")
    run = pl.kernel(
        _sc_body,
        out_type=jax.ShapeDtypeStruct((2 * _NROWS * _ROW,), x.dtype),
        mesh=mesh,
        scratch_types=[
            pltpu.VMEM(((_KR + 1) * _ROW,), jnp.float32),
            pltpu.VMEM((_KR * _ROW,), jnp.float32),
            pltpu.VMEM((_ROW,), jnp.int32),
            pltpu.SemaphoreType.DMA((3,)),
        ],
    )
    out = run(x_flat, sel)
    return out.reshape(2, N, C, T, H, W)


# speedup vs baseline: 2.2856x; 2.2856x over previous
import jax
import jax.numpy as jnp
from jax import lax
from jax.experimental import pallas as pl
from jax.experimental.pallas import tpu as pltpu
from jax.experimental.pallas import tpu_sc as plsc

_N, _C, _T, _HW = 8, 256, 32, 196
_ROW = _T * _HW
_NROWS = _N * _C
_NW = 32
_KR = 8
_CHUNKS_PER_W = _NROWS // (_NW * _KR)
_G = _ROW // 16
_GCOPY = (_T - 1) * _HW // 16 + 1


def _sc_body(x_hbm, sel_hbm, out_hbm, buf, buf2, selv, sems):
    wid = lax.axis_index("s") * 2 + lax.axis_index("c")
    pltpu.sync_copy(sel_hbm, selv)
    wrow = wid * (_NROWS // _NW)

    def chunk_step(k, carry):
        r0 = wrow + k * _KR

        @pl.when(k > 0)
        def _():
            pltpu.make_async_copy(
                buf.at[:, : _ROW], out_hbm.at[0, pl.ds(r0, _KR)], sems.at[1]
            ).wait()

        cin = pltpu.make_async_copy(
            x_hbm.at[pl.ds(r0, _KR)], buf.at[:, : _ROW], sems.at[0]
        )
        cin.start()
        cin.wait()
        c0 = pltpu.make_async_copy(
            buf.at[:, : _ROW], out_hbm.at[0, pl.ds(r0, _KR)], sems.at[1]
        )
        c0.start()

        @pl.when(k > 0)
        def _():
            pltpu.make_async_copy(
                buf2, out_hbm.at[1, pl.ds(r0, _KR)], sems.at[2]
            ).wait()

        def row_step(r, c):
            for g in range(_G):
                col = 16 * g
                a = buf[r, pl.ds(col, 16)]
                if g >= _GCOPY:
                    buf2[r, pl.ds(col, 16)] = a
                else:
                    b = buf[r, pl.ds(col + _HW, 16)]
                    s = selv[pl.ds(col, 16)]
                    buf2[r, pl.ds(col, 16)] = jnp.where(s != 0, a, b)
            return c

        lax.fori_loop(0, _KR, row_step, 0)
        c1 = pltpu.make_async_copy(
            buf2, out_hbm.at[1, pl.ds(r0, _KR)], sems.at[2]
        )
        c1.start()
        return carry

    lax.fori_loop(0, _CHUNKS_PER_W, chunk_step, 0)
    last = wrow + (_CHUNKS_PER_W - 1) * _KR
    pltpu.make_async_copy(
        buf.at[:, : _ROW], out_hbm.at[0, pl.ds(last, _KR)], sems.at[1]
    ).wait()
    pltpu.make_async_copy(
        buf2, out_hbm.at[1, pl.ds(last, _KR)], sems.at[2]
    ).wait()


def kernel(x, t_length):
    N, C, T, H, W = x.shape
    idx = jnp.mod(t_length.astype(jnp.int32) - 1, T)
    keep = jnp.zeros((T,), jnp.int32).at[idx].set(1).at[T - 1].set(1)
    sel = jnp.repeat(keep, H * W)
    x2 = x.reshape(_NROWS, _ROW)

    mesh = plsc.VectorSubcoreMesh(core_axis_name="c", subcore_axis_name="s")
    run = pl.kernel(
        _sc_body,
        out_type=jax.ShapeDtypeStruct((2, _NROWS, _ROW), x.dtype),
        mesh=mesh,
        scratch_types=[
            pltpu.VMEM((_KR, _ROW + 128), jnp.float32),
            pltpu.VMEM((_KR, _ROW), jnp.float32),
            pltpu.VMEM((_ROW,), jnp.int32),
            pltpu.SemaphoreType.DMA((3,)),
        ],
    )
    out = run(x2, sel)
    return out.reshape(2, N, C, T, H, W)


# speedup vs baseline: 3.8189x; 1.6708x over previous
import jax
import jax.numpy as jnp
from jax import lax
from jax.experimental import pallas as pl
from jax.experimental.pallas import tpu as pltpu

_NROWS = 2048
_ROW = 6272
_HW = 196
_BR = 128
_STEPS = _NROWS // _BR
_D = 3


def _ts_kernel(x_ref, sel_ref, o_ref, xs, sh, sems):
    k = pl.program_id(0)
    slot = lax.rem(k, _D)

    @pl.when(k >= _D)
    def _():
        pltpu.make_async_copy(
            xs.at[slot], o_ref.at[0, pl.ds(0, _BR)], sems.at[2 * slot]
        ).wait()
        pltpu.make_async_copy(
            sh.at[slot], o_ref.at[1, pl.ds(0, _BR)], sems.at[2 * slot + 1]
        ).wait()

    xv = x_ref[...]
    xs[slot] = xv
    shifted = jnp.concatenate([xv[:, _HW:], xv[:, _ROW - _HW :]], axis=1)
    sh[slot] = jnp.where(sel_ref[...] != 0, xv, shifted)

    base = k * _BR
    pltpu.make_async_copy(
        xs.at[slot], o_ref.at[0, pl.ds(base, _BR)], sems.at[2 * slot]
    ).start()
    pltpu.make_async_copy(
        sh.at[slot], o_ref.at[1, pl.ds(base, _BR)], sems.at[2 * slot + 1]
    ).start()

    @pl.when(k == _STEPS - 1)
    def _():
        for s in range(_D):
            pltpu.make_async_copy(
                xs.at[s], o_ref.at[0, pl.ds(0, _BR)], sems.at[2 * s]
            ).wait()
            pltpu.make_async_copy(
                sh.at[s], o_ref.at[1, pl.ds(0, _BR)], sems.at[2 * s + 1]
            ).wait()


def kernel(x, t_length):
    N, C, T, H, W = x.shape
    idx = jnp.mod(t_length.astype(jnp.int32) - 1, T)
    keep = jnp.zeros((T,), jnp.float32).at[idx].set(1.0)
    sel = jnp.repeat(keep, H * W).reshape(1, _ROW)
    x2 = x.reshape(_NROWS, _ROW)

    out = pl.pallas_call(
        _ts_kernel,
        grid=(_STEPS,),
        in_specs=[
            pl.BlockSpec((_BR, _ROW), lambda k: (k, 0)),
            pl.BlockSpec((1, _ROW), lambda k: (0, 0)),
        ],
        out_specs=pl.BlockSpec(memory_space=pl.ANY),
        out_shape=jax.ShapeDtypeStruct((2, _NROWS, _ROW), x.dtype),
        scratch_shapes=[
            pltpu.VMEM((_D, _BR, _ROW), jnp.float32),
            pltpu.VMEM((_D, _BR, _ROW), jnp.float32),
            pltpu.SemaphoreType.DMA((2 * _D,)),
        ],
    )(x2, sel)
    return out.reshape(2, N, C, T, H, W)


# speedup vs baseline: 7.1814x; 1.8805x over previous
import jax
import jax.numpy as jnp
from jax.experimental import pallas as pl
from jax.experimental.pallas import tpu as pltpu

_CB = 256


def _shift_kernel(idx_ref, x_ref, o_ref):
    T = x_ref.shape[2]
    o_ref[0] = x_ref[...]
    o_ref[1, :, :, : T - 1] = x_ref[:, :, 1:]
    o_ref[1, :, :, T - 1 :] = x_ref[:, :, T - 1 :]
    for n in range(idx_ref.shape[0]):
        i = idx_ref[n]
        o_ref[1, :, :, pl.ds(i, 1)] = x_ref[:, :, pl.ds(i, 1)]


def kernel(x, t_length):
    N, C, T, H, W = x.shape
    HW = H * W
    idx = jnp.mod(t_length.astype(jnp.int32) - 1, T)
    xr = x.reshape(N, C, T, HW)

    def in_map(n, c, iref):
        return (n, c, 0, 0)

    def out_map(n, c, iref):
        return (0, n, c, 0, 0)

    out = pl.pallas_call(
        _shift_kernel,
        grid_spec=pltpu.PrefetchScalarGridSpec(
            num_scalar_prefetch=1,
            grid=(N, C // _CB),
            in_specs=[pl.BlockSpec((1, _CB, T, HW), in_map)],
            out_specs=pl.BlockSpec((2, 1, _CB, T, HW), out_map),
        ),
        out_shape=jax.ShapeDtypeStruct((2, N, C, T, HW), x.dtype),
    )(idx, xr)
    return out.reshape(2, N, C, T, H, W)
